# baseline (speedup 1.0000x reference)
import functools

import jax
import jax.numpy as jnp
from jax import lax
from jax.experimental import pallas as pl
from jax.experimental.pallas import tpu as pltpu
from jax.experimental.pallas import tpu_sc as plsc

_B, _V, _D = 1024, 1000, 32
_NC, _NS = 2, 16
_NW = _NC * _NS
_RPW = _B // _NW
_RG = 8
_VFULL = _V // 16


def _bcast(vec, j):
    idx = jnp.full((16, 1), j, dtype=jnp.int32)
    return lax.gather(
        vec, idx,
        dimension_numbers=lax.GatherDimensionNumbers(
            offset_dims=(), collapsed_slice_dims=(0,), start_index_map=(0,)),
        slice_sizes=(1,),
        mode=lax.GatherScatterMode.PROMISE_IN_BOUNDS)


@functools.partial(
    pl.kernel,
    mesh=plsc.VectorSubcoreMesh(core_axis_name="c", subcore_axis_name="s"),
    out_type=jax.ShapeDtypeStruct((_B, _D), jnp.float32),
    scratch_types=[
        pltpu.VMEM((_RPW, _V), jnp.float32),
        pltpu.VMEM((_V // 4, _D * 4), jnp.float32),
        pltpu.VMEM((_RPW, _D), jnp.float32),
        pltpu.SemaphoreType.DMA,
        pltpu.SemaphoreType.DMA,
    ],
)
def _sc_bag_kernel(x_hbm, e_hbm, out_hbm, x_v, e_v, o_v, sem_e, sem_x):
    wid = lax.axis_index("s") * _NC + lax.axis_index("c")
    base = wid * _RPW
    cp_e = pltpu.async_copy(e_hbm, e_v, sem_e)
    cp_x = pltpu.async_copy(x_hbm.at[pl.ds(base, _RPW), :], x_v, sem_x)
    cp_e.wait()
    cp_x.wait()

    for rg in range(_RPW // _RG):
        rows = [rg * _RG + k for k in range(_RG)]

        def chunk(start, accs, j_lo=0):
            sdiv = start // 4
            xc = [x_v[r, pl.ds(start, 16)] for r in rows]
            accs = list(accs)
            for j in range(j_lo, 16):
                e0 = e_v[sdiv + j // 4, pl.ds((j % 4) * 32, 16)]
                e1 = e_v[sdiv + j // 4, pl.ds((j % 4) * 32 + 16, 16)]
                for k in range(_RG):
                    w = _bcast(xc[k], j)
                    accs[2 * k] = accs[2 * k] + w * e0
                    accs[2 * k + 1] = accs[2 * k + 1] + w * e1
            return tuple(accs)

        init = tuple(jnp.zeros((16,), jnp.float32) for _ in range(2 * _RG))
        accs = lax.fori_loop(0, _VFULL,
                             lambda vc, a: chunk(vc * 16, a), init)
        accs = chunk(_V - 16, accs, j_lo=8)
        for k in range(_RG):
            o_v[rows[k], pl.ds(0, 16)] = accs[2 * k]
            o_v[rows[k], pl.ds(16, 16)] = accs[2 * k + 1]

    pltpu.sync_copy(o_v, out_hbm.at[pl.ds(base, _RPW), :])


def kernel(inputs, embeddings):
    e_packed = embeddings.reshape(_V // 4, _D * 4)
    return _sc_bag_kernel(inputs, e_packed)


# speedup vs baseline: 13.5423x; 13.5423x over previous
import jax
import jax.numpy as jnp
from jax.experimental import pallas as pl

_BB = 512


def _bag_matmul_kernel(xa_ref, xb_ref, e_ref, o_ref):
    h = _BB // 2
    o_ref[:h, :] = jnp.dot(xa_ref[...], e_ref[...],
                           preferred_element_type=jnp.float32)
    o_ref[h:, :] = jnp.dot(xb_ref[...], e_ref[...],
                           preferred_element_type=jnp.float32)


def kernel(inputs, embeddings):
    B, V = inputs.shape
    _, D = embeddings.shape
    h = _BB // 2
    return pl.pallas_call(
        _bag_matmul_kernel,
        grid=(B // _BB,),
        in_specs=[
            pl.BlockSpec((h, V), lambda i: (2 * i, 0)),
            pl.BlockSpec((h, V), lambda i: (2 * i + 1, 0)),
            pl.BlockSpec((V, D), lambda i: (0, 0)),
        ],
        out_specs=pl.BlockSpec((_BB, D), lambda i: (i, 0)),
        out_shape=jax.ShapeDtypeStruct((B, D), jnp.float32),
    )(inputs, inputs, embeddings)
